# baseline (speedup 1.0000x reference)
import functools

import jax
import jax.numpy as jnp
from jax import lax
from jax.experimental import pallas as pl
from jax.experimental.pallas import tpu as pltpu
from jax.experimental.pallas import tpu_sc as plsc

VOCAB = 1000000
EMBED = 64
BATCH = 16384
HIST = 50

NW = 32
TOTAL = BATCH * HIST
C = 128
NCHUNK_TOTAL = TOTAL // C
NCHUNK = NCHUNK_TOTAL // NW
GB = 4
NGROUP = NCHUNK // GB
BG = BATCH // C


@functools.partial(
    pl.kernel,
    mesh=plsc.VectorSubcoreMesh(core_axis_name="c", subcore_axis_name="s"),
    out_type=jax.ShapeDtypeStruct((HIST, 8, BG, 8, C), jnp.float32),
    scratch_types=[
        pltpu.VMEM((NCHUNK, C), jnp.int32),
        pltpu.VMEM((GB, C, EMBED), jnp.float32),
        pltpu.VMEM((2, 8, GB, 8, C), jnp.float32),
        pltpu.SemaphoreType.DMA,
        pltpu.SemaphoreType.DMA,
        pltpu.SemaphoreType.DMA,
        pltpu.SemaphoreType.DMA,
        pltpu.SemaphoreType.DMA,
        pltpu.SemaphoreType.DMA,
    ],
    compiler_params=pltpu.CompilerParams(
        use_tc_tiling_on_sc=False, needs_layout_passes=False
    ),
)
def _gather_rows(idx_hbm, table_hbm, out_hbm,
                 idx_v, rows_v, tr_v, g0, g1, g2, g3, s0, s1):
    gsem = [g0, g1, g2, g3]
    ssem = [s0, s1]
    cid = lax.axis_index("c")
    sid = lax.axis_index("s")
    wid = sid * 2 + cid
    pltpu.sync_copy(idx_hbm.at[wid], idx_v)
    bg0 = wid * GB

    lane = lax.iota(jnp.int32, 16)
    row_bases = [g * 16 + lane for g in range(8)]

    def fire_gather(j, b):
        return pltpu.async_copy(table_hbm.at[idx_v.at[j]], rows_v.at[b], gsem[b])

    def transpose_chunk(p, b):
        rb = rows_v.at[b]

        def body(cg, carry):
            def load_burst(cs):
                col = cg * 8 + jnp.full((16,), cs, dtype=jnp.int32)
                return [
                    plsc.load_gather(rb, [row_bases[gg], col])
                    for gg in range(8)
                ]

            prev = load_burst(0)
            for cs in range(1, 8):
                cur = load_burst(cs)
                for gg in range(8):
                    tr_v[p, cg, b, cs - 1, pl.ds(gg * 16, 16)] = prev[gg]
                prev = cur
            for gg in range(8):
                tr_v[p, cg, b, 7, pl.ds(gg * 16, 16)] = prev[gg]
            return carry

        lax.fori_loop(0, 8, body, 0)

    def pair(m, carry):
        store_cps = []
        for p in range(2):
            g = 2 * m + p
            gathers = [fire_gather(g * GB + b, b) for b in range(GB)]
            for b in range(GB):
                gathers[b].wait()
                transpose_chunk(p, b)
            if p == 1:
                for cp in store_cps:
                    cp.wait()
            store_cps = [
                pltpu.async_copy(
                    tr_v.at[p, cg], out_hbm.at[g, cg, pl.ds(bg0, GB)], ssem[p]
                )
                for cg in range(8)
            ]
        for cp in store_cps:
            cp.wait()
        return carry

    lax.fori_loop(0, NGROUP // 2, pair, 0)


def kernel(data, ivectors, ovectors):
    idx = (
        data.astype(jnp.int32).T
        .reshape(HIST, NW, GB, C)
        .transpose(1, 0, 2, 3)
        .reshape(NW, NCHUNK, C)
    )
    out5 = _gather_rows(idx, ivectors)
    return out5.transpose(2, 4, 0, 1, 3).reshape(BATCH, HIST, EMBED)


# speedup vs baseline: 1.3443x; 1.3443x over previous
import functools

import jax
import jax.numpy as jnp
from jax import lax
from jax.experimental import pallas as pl
from jax.experimental.pallas import tpu as pltpu
from jax.experimental.pallas import tpu_sc as plsc

VOCAB = 1000000
EMBED = 64
BATCH = 16384
HIST = 50

NW = 32
TOTAL = BATCH * HIST
R_PER_W = TOTAL // NW
C = 128
NCHUNK = R_PER_W // C
K = 8
NGROUP = NCHUNK // K


@functools.partial(
    pl.kernel,
    mesh=plsc.VectorSubcoreMesh(core_axis_name="c", subcore_axis_name="s"),
    out_type=jax.ShapeDtypeStruct((TOTAL, EMBED), jnp.float32),
    scratch_types=[
        pltpu.VMEM((NCHUNK, C), jnp.int32),
        pltpu.VMEM((K, C, EMBED), jnp.float32),
        pltpu.SemaphoreType.DMA,
        pltpu.SemaphoreType.DMA,
    ],
    compiler_params=pltpu.CompilerParams(use_tc_tiling_on_sc=False),
)
def _gather_rows(idx_hbm, table_hbm, out_hbm, idx_v, rows_v, gsem, ssem):
    cid = lax.axis_index("c")
    sid = lax.axis_index("s")
    wid = sid * 2 + cid
    pltpu.sync_copy(idx_hbm.at[wid], idx_v)
    base = wid * R_PER_W

    def group(g, carry):
        j0 = g * K
        gathers = []
        for b in range(K):
            gathers.append(
                pltpu.async_copy(table_hbm.at[idx_v.at[j0 + b]], rows_v.at[b], gsem)
            )
        stores = []
        for b in range(K):
            gathers[b].wait()
            stores.append(
                pltpu.async_copy(
                    rows_v.at[b], out_hbm.at[pl.ds(base + (j0 + b) * C, C)], ssem
                )
            )
        for b in range(K):
            stores[b].wait()
        return carry

    lax.fori_loop(0, NGROUP, group, 0)


def kernel(data, ivectors, ovectors):
    idx = data.reshape(TOTAL).astype(jnp.int32).reshape(NW, NCHUNK, C)
    flat = _gather_rows(idx, ivectors)
    return flat.reshape(BATCH, HIST, EMBED)
